# baseline (speedup 1.0000x reference)
import functools

import jax
import jax.numpy as jnp
from jax import lax
from jax.experimental import pallas as pl
from jax.experimental.pallas import tpu as pltpu
from jax.experimental.pallas import tpu_sc as plsc

_MARGIN = 0.5
_N = 16384
_NC = 2
_NS = 16
_L = 16
_NW = _NC * _NS
_PER_W = _N // _NW
_CH = 128
_NCH = _PER_W // _CH

_ROWS_SC = 4096
_SC_E = _ROWS_SC * _N
_SC_BASE = (_N - _ROWS_SC) * _N
_PT = _SC_E // _NW
_CE = 32768
_NMC = _PT // _CE
_ACCS = 8


def _sc_body(rows_hbm, cols_hbm, tflat_hbm, out_hbm,
             rows_v, cols_v, idx_v, vals_v, acc_v, buf0, buf1,
             gsem, sem0, sem1):
    wid = lax.axis_index("s") * _NC + lax.axis_index("c")
    base = wid * _PER_W
    pltpu.sync_copy(rows_hbm.at[pl.ds(base, _PER_W)], rows_v)
    pltpu.sync_copy(cols_hbm.at[pl.ds(base, _PER_W)], cols_v)
    for c in range(_NCH):
        for j in range(_CH // _L):
            o = c * _CH + j * _L
            r = rows_v[pl.ds(o, _L)]
            q = cols_v[pl.ds(o, _L)]
            f = (lax.shift_right_logical(r, 3) * 131072
                 + lax.shift_right_logical(q, 7) * 1024
                 + lax.bitwise_and(r, 7) * 128
                 + lax.bitwise_and(q, 127))
            idx_v[c, pl.ds(j * _L, _L)] = f
    copies = [
        pltpu.async_copy(tflat_hbm.at[idx_v.at[c]], vals_v.at[c], gsem)
        for c in range(_NCH)
    ]
    for cp in copies:
        cp.wait()
    acc = jnp.zeros((_L,), jnp.float32)
    for c in range(_NCH):
        for j in range(_CH // _L):
            acc = acc + vals_v[c, pl.ds(j * _L, _L)]
    acc_v[...] = acc
    pltpu.sync_copy(acc_v, out_hbm.at[0, wid])

    mbase = _SC_BASE + wid * _PT

    def _chunk_sum(buf, accs):
        def inner(j, accs):
            o = j * (_ACCS * _L)
            return tuple(
                accs[i] + buf[pl.ds(o + i * _L, _L)] for i in range(_ACCS)
            )
        return lax.fori_loop(0, _CE // (_ACCS * _L), inner, accs)

    def outer(kk, accs):
        pltpu.make_async_copy(tflat_hbm.at[pl.ds(0, _CE)], buf0, sem0).wait()

        @pl.when(kk * 2 + 1 < _NMC)
        def _():
            pltpu.async_copy(
                tflat_hbm.at[pl.ds(mbase + (kk * 2 + 1) * _CE, _CE)], buf1, sem1)

        accs = _chunk_sum(buf0, accs)
        pltpu.make_async_copy(tflat_hbm.at[pl.ds(0, _CE)], buf1, sem1).wait()

        @pl.when(kk < _NMC // 2 - 1)
        def _():
            pltpu.async_copy(
                tflat_hbm.at[pl.ds(mbase + (kk * 2 + 2) * _CE, _CE)], buf0, sem0)

        accs = _chunk_sum(buf1, accs)
        return accs

    pltpu.async_copy(tflat_hbm.at[pl.ds(mbase, _CE)], buf0, sem0)
    accs0 = tuple(jnp.zeros((_L,), jnp.float32) for _ in range(_ACCS))
    accs = lax.fori_loop(0, _NMC // 2, outer, accs0)
    macc = accs[0]
    for i in range(1, _ACCS):
        macc = macc + accs[i]
    acc_v[...] = macc
    pltpu.sync_copy(acc_v, out_hbm.at[1, wid])


_sc_gather = functools.partial(
    pl.kernel,
    out_type=jax.ShapeDtypeStruct((2, _NW, _L), jnp.float32),
    mesh=plsc.VectorSubcoreMesh(core_axis_name="c", subcore_axis_name="s"),
    scratch_types=[
        pltpu.VMEM((_PER_W,), jnp.int32),
        pltpu.VMEM((_PER_W,), jnp.int32),
        pltpu.VMEM((_NCH, _CH), jnp.int32),
        pltpu.VMEM((_NCH, _CH), jnp.float32),
        pltpu.VMEM((_L,), jnp.float32),
        pltpu.VMEM((_CE,), jnp.float32),
        pltpu.VMEM((_CE,), jnp.float32),
        pltpu.SemaphoreType.DMA,
        pltpu.SemaphoreType.DMA,
        pltpu.SemaphoreType.DMA,
    ],
)(_sc_body)


_BM = 256
_TC_ROWS = _N - _ROWS_SC
_GRID = _TC_ROWS // _BM


def _tc_sum_body(x_ref, out_ref, acc_ref):
    i = pl.program_id(0)

    @pl.when(i == 0)
    def _():
        acc_ref[...] = jnp.zeros_like(acc_ref)

    acc_ref[...] += jnp.sum(x_ref[...].reshape(_BM // 8, 8, _N), axis=0)

    @pl.when(i == _GRID - 1)
    def _():
        out_ref[0] = jnp.sum(acc_ref[...])


_tc_sum = pl.pallas_call(
    _tc_sum_body,
    grid=(_GRID,),
    in_specs=[pl.BlockSpec((_BM, _N), lambda i: (i, 0))],
    out_specs=pl.BlockSpec(memory_space=pltpu.SMEM),
    out_shape=jax.ShapeDtypeStruct((1,), jnp.float32),
    scratch_shapes=[pltpu.VMEM((8, _N), jnp.float32)],
)


def _combine_body(tot_ref, parts_ref, out_ref):
    pos_sum = jnp.sum(parts_ref[0])
    mat_sum = tot_ref[0] + jnp.sum(parts_ref[1])
    out_ref[0] = (mat_sum / (_N * _N)) - (pos_sum / _N) + _MARGIN


_combine = pl.pallas_call(
    _combine_body,
    in_specs=[
        pl.BlockSpec(memory_space=pltpu.SMEM),
        pl.BlockSpec((2, _NW, _L), lambda: (0, 0, 0)),
    ],
    out_specs=pl.BlockSpec(memory_space=pltpu.SMEM),
    out_shape=jax.ShapeDtypeStruct((1,), jnp.float32),
)


def kernel(match_pair, similarity_matrix):
    mp = match_pair.astype(jnp.int32)
    tflat = (similarity_matrix
             .reshape(2048, 8, 128, 128)
             .transpose(0, 2, 1, 3)
             .reshape(_N * _N))
    partials = _sc_gather(mp[0], mp[1], tflat)
    total = _tc_sum(similarity_matrix)
    loss = _combine(total, partials)
    return loss[0]


# speedup vs baseline: 1.0062x; 1.0062x over previous
import functools

import jax
import jax.numpy as jnp
from jax import lax
from jax.experimental import pallas as pl
from jax.experimental.pallas import tpu as pltpu
from jax.experimental.pallas import tpu_sc as plsc

_MARGIN = 0.5
_N = 16384
_NC = 2
_NS = 16
_L = 16
_NW = _NC * _NS
_PER_W = _N // _NW
_CH = 128
_NCH = _PER_W // _CH

_ROWS_SC = 3072
_SC_E = _ROWS_SC * _N
_SC_BASE = (_N - _ROWS_SC) * _N
_PT = _SC_E // _NW
_CE = 16384
_NMC = _PT // _CE
_ACCS = 8


def _sc_body(rows_hbm, cols_hbm, tflat_hbm, out_hbm,
             rows_v, cols_v, idx_v, vals_v, acc_v, buf0, buf1,
             gsem, sem0, sem1):
    wid = lax.axis_index("s") * _NC + lax.axis_index("c")
    base = wid * _PER_W
    pltpu.sync_copy(rows_hbm.at[pl.ds(base, _PER_W)], rows_v)
    pltpu.sync_copy(cols_hbm.at[pl.ds(base, _PER_W)], cols_v)
    for c in range(_NCH):
        for j in range(_CH // _L):
            o = c * _CH + j * _L
            r = rows_v[pl.ds(o, _L)]
            q = cols_v[pl.ds(o, _L)]
            f = (lax.shift_right_logical(r, 3) * 131072
                 + lax.shift_right_logical(q, 7) * 1024
                 + lax.bitwise_and(r, 7) * 128
                 + lax.bitwise_and(q, 127))
            idx_v[c, pl.ds(j * _L, _L)] = f
    copies = [
        pltpu.async_copy(tflat_hbm.at[idx_v.at[c]], vals_v.at[c], gsem)
        for c in range(_NCH)
    ]
    for cp in copies:
        cp.wait()
    acc = jnp.zeros((_L,), jnp.float32)
    for c in range(_NCH):
        for j in range(_CH // _L):
            acc = acc + vals_v[c, pl.ds(j * _L, _L)]
    acc_v[...] = acc
    pltpu.sync_copy(acc_v, out_hbm.at[0, wid])

    mbase = _SC_BASE + wid * _PT

    def _chunk_sum(buf, accs):
        def inner(j, accs):
            o = j * (_ACCS * _L)
            return tuple(
                accs[i] + buf[pl.ds(o + i * _L, _L)] for i in range(_ACCS)
            )
        return lax.fori_loop(0, _CE // (_ACCS * _L), inner, accs)

    def outer(kk, accs):
        pltpu.make_async_copy(tflat_hbm.at[pl.ds(0, _CE)], buf0, sem0).wait()

        @pl.when(kk * 2 + 1 < _NMC)
        def _():
            pltpu.async_copy(
                tflat_hbm.at[pl.ds(mbase + (kk * 2 + 1) * _CE, _CE)], buf1, sem1)

        accs = _chunk_sum(buf0, accs)
        pltpu.make_async_copy(tflat_hbm.at[pl.ds(0, _CE)], buf1, sem1).wait()

        @pl.when(kk < _NMC // 2 - 1)
        def _():
            pltpu.async_copy(
                tflat_hbm.at[pl.ds(mbase + (kk * 2 + 2) * _CE, _CE)], buf0, sem0)

        accs = _chunk_sum(buf1, accs)
        return accs

    pltpu.async_copy(tflat_hbm.at[pl.ds(mbase, _CE)], buf0, sem0)
    accs0 = tuple(jnp.zeros((_L,), jnp.float32) for _ in range(_ACCS))
    accs = lax.fori_loop(0, _NMC // 2, outer, accs0)
    macc = accs[0]
    for i in range(1, _ACCS):
        macc = macc + accs[i]
    acc_v[...] = macc
    pltpu.sync_copy(acc_v, out_hbm.at[1, wid])


_sc_gather = functools.partial(
    pl.kernel,
    out_type=jax.ShapeDtypeStruct((2, _NW, _L), jnp.float32),
    mesh=plsc.VectorSubcoreMesh(core_axis_name="c", subcore_axis_name="s"),
    scratch_types=[
        pltpu.VMEM((_PER_W,), jnp.int32),
        pltpu.VMEM((_PER_W,), jnp.int32),
        pltpu.VMEM((_NCH, _CH), jnp.int32),
        pltpu.VMEM((_NCH, _CH), jnp.float32),
        pltpu.VMEM((_L,), jnp.float32),
        pltpu.VMEM((_CE,), jnp.float32),
        pltpu.VMEM((_CE,), jnp.float32),
        pltpu.SemaphoreType.DMA,
        pltpu.SemaphoreType.DMA,
        pltpu.SemaphoreType.DMA,
    ],
)(_sc_body)


_BM = 256
_TC_ROWS = _N - _ROWS_SC
_GRID = _TC_ROWS // _BM


def _tc_sum_body(x_ref, out_ref, acc_ref):
    i = pl.program_id(0)

    @pl.when(i == 0)
    def _():
        acc_ref[...] = jnp.zeros_like(acc_ref)

    acc_ref[...] += jnp.sum(x_ref[...].reshape(_BM // 8, 8, _N), axis=0)

    @pl.when(i == _GRID - 1)
    def _():
        out_ref[0] = jnp.sum(acc_ref[...])


_tc_sum = pl.pallas_call(
    _tc_sum_body,
    grid=(_GRID,),
    in_specs=[pl.BlockSpec((_BM, _N), lambda i: (i, 0))],
    out_specs=pl.BlockSpec(memory_space=pltpu.SMEM),
    out_shape=jax.ShapeDtypeStruct((1,), jnp.float32),
    scratch_shapes=[pltpu.VMEM((8, _N), jnp.float32)],
)


def _combine_body(tot_ref, parts_ref, out_ref):
    pos_sum = jnp.sum(parts_ref[0])
    mat_sum = tot_ref[0] + jnp.sum(parts_ref[1])
    out_ref[0] = (mat_sum / (_N * _N)) - (pos_sum / _N) + _MARGIN


_combine = pl.pallas_call(
    _combine_body,
    in_specs=[
        pl.BlockSpec(memory_space=pltpu.SMEM),
        pl.BlockSpec((2, _NW, _L), lambda: (0, 0, 0)),
    ],
    out_specs=pl.BlockSpec(memory_space=pltpu.SMEM),
    out_shape=jax.ShapeDtypeStruct((1,), jnp.float32),
)


def kernel(match_pair, similarity_matrix):
    mp = match_pair.astype(jnp.int32)
    tflat = (similarity_matrix
             .reshape(2048, 8, 128, 128)
             .transpose(0, 2, 1, 3)
             .reshape(_N * _N))
    partials = _sc_gather(mp[0], mp[1], tflat)
    total = _tc_sum(similarity_matrix)
    loss = _combine(total, partials)
    return loss[0]


# speedup vs baseline: 1.0137x; 1.0075x over previous
import functools

import jax
import jax.numpy as jnp
from jax import lax
from jax.experimental import pallas as pl
from jax.experimental.pallas import tpu as pltpu
from jax.experimental.pallas import tpu_sc as plsc

_MARGIN = 0.5
_N = 16384
_NC = 2
_NS = 16
_L = 16
_NW = _NC * _NS
_PER_W = _N // _NW
_CH = 128
_NCH = _PER_W // _CH

_ROWS_SC = 1024
_SC_E = _ROWS_SC * _N
_SC_BASE = (_N - _ROWS_SC) * _N
_PT = _SC_E // _NW
_CE = 16384
_NMC = _PT // _CE
_ACCS = 8


def _sc_body(rows_hbm, cols_hbm, tflat_hbm, out_hbm,
             rows_v, cols_v, idx_v, vals_v, acc_v, buf0, buf1,
             gsem, sem0, sem1):
    wid = lax.axis_index("s") * _NC + lax.axis_index("c")
    base = wid * _PER_W
    pltpu.sync_copy(rows_hbm.at[pl.ds(base, _PER_W)], rows_v)
    pltpu.sync_copy(cols_hbm.at[pl.ds(base, _PER_W)], cols_v)
    for c in range(_NCH):
        for j in range(_CH // _L):
            o = c * _CH + j * _L
            r = rows_v[pl.ds(o, _L)]
            q = cols_v[pl.ds(o, _L)]
            f = (lax.shift_right_logical(r, 3) * 131072
                 + lax.shift_right_logical(q, 7) * 1024
                 + lax.bitwise_and(r, 7) * 128
                 + lax.bitwise_and(q, 127))
            idx_v[c, pl.ds(j * _L, _L)] = f
    copies = [
        pltpu.async_copy(tflat_hbm.at[idx_v.at[c]], vals_v.at[c], gsem)
        for c in range(_NCH)
    ]
    for cp in copies:
        cp.wait()
    acc = jnp.zeros((_L,), jnp.float32)
    for c in range(_NCH):
        for j in range(_CH // _L):
            acc = acc + vals_v[c, pl.ds(j * _L, _L)]
    acc_v[...] = acc
    pltpu.sync_copy(acc_v, out_hbm.at[0, wid])

    mbase = _SC_BASE + wid * _PT

    def _chunk_sum(buf, accs):
        def inner(j, accs):
            o = j * (_ACCS * _L)
            return tuple(
                accs[i] + buf[pl.ds(o + i * _L, _L)] for i in range(_ACCS)
            )
        return lax.fori_loop(0, _CE // (_ACCS * _L), inner, accs)

    def outer(kk, accs):
        pltpu.make_async_copy(tflat_hbm.at[pl.ds(0, _CE)], buf0, sem0).wait()

        @pl.when(kk * 2 + 1 < _NMC)
        def _():
            pltpu.async_copy(
                tflat_hbm.at[pl.ds(mbase + (kk * 2 + 1) * _CE, _CE)], buf1, sem1)

        accs = _chunk_sum(buf0, accs)
        pltpu.make_async_copy(tflat_hbm.at[pl.ds(0, _CE)], buf1, sem1).wait()

        @pl.when(kk < _NMC // 2 - 1)
        def _():
            pltpu.async_copy(
                tflat_hbm.at[pl.ds(mbase + (kk * 2 + 2) * _CE, _CE)], buf0, sem0)

        accs = _chunk_sum(buf1, accs)
        return accs

    pltpu.async_copy(tflat_hbm.at[pl.ds(mbase, _CE)], buf0, sem0)
    accs0 = tuple(jnp.zeros((_L,), jnp.float32) for _ in range(_ACCS))
    accs = lax.fori_loop(0, _NMC // 2, outer, accs0)
    macc = accs[0]
    for i in range(1, _ACCS):
        macc = macc + accs[i]
    acc_v[...] = macc
    pltpu.sync_copy(acc_v, out_hbm.at[1, wid])


_sc_gather = functools.partial(
    pl.kernel,
    out_type=jax.ShapeDtypeStruct((2, _NW, _L), jnp.float32),
    mesh=plsc.VectorSubcoreMesh(core_axis_name="c", subcore_axis_name="s"),
    scratch_types=[
        pltpu.VMEM((_PER_W,), jnp.int32),
        pltpu.VMEM((_PER_W,), jnp.int32),
        pltpu.VMEM((_NCH, _CH), jnp.int32),
        pltpu.VMEM((_NCH, _CH), jnp.float32),
        pltpu.VMEM((_L,), jnp.float32),
        pltpu.VMEM((_CE,), jnp.float32),
        pltpu.VMEM((_CE,), jnp.float32),
        pltpu.SemaphoreType.DMA,
        pltpu.SemaphoreType.DMA,
        pltpu.SemaphoreType.DMA,
    ],
)(_sc_body)


_BM = 256
_TC_ROWS = _N - _ROWS_SC
_GRID = _TC_ROWS // _BM


def _tc_sum_body(x_ref, out_ref, acc_ref):
    i = pl.program_id(0)

    @pl.when(i == 0)
    def _():
        acc_ref[...] = jnp.zeros_like(acc_ref)

    acc_ref[...] += jnp.sum(x_ref[...].reshape(_BM // 8, 8, _N), axis=0)

    @pl.when(i == _GRID - 1)
    def _():
        out_ref[0] = jnp.sum(acc_ref[...])


_tc_sum = pl.pallas_call(
    _tc_sum_body,
    grid=(_GRID,),
    in_specs=[pl.BlockSpec((_BM, _N), lambda i: (i, 0))],
    out_specs=pl.BlockSpec(memory_space=pltpu.SMEM),
    out_shape=jax.ShapeDtypeStruct((1,), jnp.float32),
    scratch_shapes=[pltpu.VMEM((8, _N), jnp.float32)],
)


def _combine_body(tot_ref, parts_ref, out_ref):
    pos_sum = jnp.sum(parts_ref[0])
    mat_sum = tot_ref[0] + jnp.sum(parts_ref[1])
    out_ref[0] = (mat_sum / (_N * _N)) - (pos_sum / _N) + _MARGIN


_combine = pl.pallas_call(
    _combine_body,
    in_specs=[
        pl.BlockSpec(memory_space=pltpu.SMEM),
        pl.BlockSpec((2, _NW, _L), lambda: (0, 0, 0)),
    ],
    out_specs=pl.BlockSpec(memory_space=pltpu.SMEM),
    out_shape=jax.ShapeDtypeStruct((1,), jnp.float32),
)


def kernel(match_pair, similarity_matrix):
    mp = match_pair.astype(jnp.int32)
    tflat = (similarity_matrix
             .reshape(2048, 8, 128, 128)
             .transpose(0, 2, 1, 3)
             .reshape(_N * _N))
    partials = _sc_gather(mp[0], mp[1], tflat)
    total = _tc_sum(similarity_matrix)
    loss = _combine(total, partials)
    return loss[0]


# speedup vs baseline: 1.0142x; 1.0005x over previous
import functools

import jax
import jax.numpy as jnp
from jax import lax
from jax.experimental import pallas as pl
from jax.experimental.pallas import tpu as pltpu
from jax.experimental.pallas import tpu_sc as plsc

_MARGIN = 0.5
_N = 16384
_NC = 2
_NS = 16
_L = 16
_NW = _NC * _NS
_PER_W = _N // _NW
_CH = 128
_NCH = _PER_W // _CH

_ROWS_SC = 512
_SC_E = _ROWS_SC * _N
_SC_BASE = (_N - _ROWS_SC) * _N
_PT = _SC_E // _NW
_CE = 16384
_NMC = _PT // _CE
_ACCS = 8


def _sc_body(rows_hbm, cols_hbm, tflat_hbm, out_hbm,
             rows_v, cols_v, idx_v, vals_v, acc_v, buf0, buf1,
             gsem, sem0, sem1):
    wid = lax.axis_index("s") * _NC + lax.axis_index("c")
    base = wid * _PER_W
    pltpu.sync_copy(rows_hbm.at[pl.ds(base, _PER_W)], rows_v)
    pltpu.sync_copy(cols_hbm.at[pl.ds(base, _PER_W)], cols_v)
    for c in range(_NCH):
        for j in range(_CH // _L):
            o = c * _CH + j * _L
            r = rows_v[pl.ds(o, _L)]
            q = cols_v[pl.ds(o, _L)]
            f = (lax.shift_right_logical(r, 3) * 131072
                 + lax.shift_right_logical(q, 7) * 1024
                 + lax.bitwise_and(r, 7) * 128
                 + lax.bitwise_and(q, 127))
            idx_v[c, pl.ds(j * _L, _L)] = f
    copies = [
        pltpu.async_copy(tflat_hbm.at[idx_v.at[c]], vals_v.at[c], gsem)
        for c in range(_NCH)
    ]
    for cp in copies:
        cp.wait()
    acc = jnp.zeros((_L,), jnp.float32)
    for c in range(_NCH):
        for j in range(_CH // _L):
            acc = acc + vals_v[c, pl.ds(j * _L, _L)]
    acc_v[...] = acc
    pltpu.sync_copy(acc_v, out_hbm.at[0, wid])

    mbase = _SC_BASE + wid * _PT

    def _chunk_sum(buf, accs):
        def inner(j, accs):
            o = j * (_ACCS * _L)
            return tuple(
                accs[i] + buf[pl.ds(o + i * _L, _L)] for i in range(_ACCS)
            )
        return lax.fori_loop(0, _CE // (_ACCS * _L), inner, accs)

    def outer(kk, accs):
        pltpu.make_async_copy(tflat_hbm.at[pl.ds(0, _CE)], buf0, sem0).wait()

        @pl.when(kk * 2 + 1 < _NMC)
        def _():
            pltpu.async_copy(
                tflat_hbm.at[pl.ds(mbase + (kk * 2 + 1) * _CE, _CE)], buf1, sem1)

        accs = _chunk_sum(buf0, accs)
        pltpu.make_async_copy(tflat_hbm.at[pl.ds(0, _CE)], buf1, sem1).wait()

        @pl.when(kk < _NMC // 2 - 1)
        def _():
            pltpu.async_copy(
                tflat_hbm.at[pl.ds(mbase + (kk * 2 + 2) * _CE, _CE)], buf0, sem0)

        accs = _chunk_sum(buf1, accs)
        return accs

    pltpu.async_copy(tflat_hbm.at[pl.ds(mbase, _CE)], buf0, sem0)
    accs0 = tuple(jnp.zeros((_L,), jnp.float32) for _ in range(_ACCS))
    accs = lax.fori_loop(0, _NMC // 2, outer, accs0)
    macc = accs[0]
    for i in range(1, _ACCS):
        macc = macc + accs[i]
    acc_v[...] = macc
    pltpu.sync_copy(acc_v, out_hbm.at[1, wid])


_sc_gather = functools.partial(
    pl.kernel,
    out_type=jax.ShapeDtypeStruct((2, _NW, _L), jnp.float32),
    mesh=plsc.VectorSubcoreMesh(core_axis_name="c", subcore_axis_name="s"),
    scratch_types=[
        pltpu.VMEM((_PER_W,), jnp.int32),
        pltpu.VMEM((_PER_W,), jnp.int32),
        pltpu.VMEM((_NCH, _CH), jnp.int32),
        pltpu.VMEM((_NCH, _CH), jnp.float32),
        pltpu.VMEM((_L,), jnp.float32),
        pltpu.VMEM((_CE,), jnp.float32),
        pltpu.VMEM((_CE,), jnp.float32),
        pltpu.SemaphoreType.DMA,
        pltpu.SemaphoreType.DMA,
        pltpu.SemaphoreType.DMA,
    ],
)(_sc_body)


_BM = 256
_TC_ROWS = _N - _ROWS_SC
_GRID = _TC_ROWS // _BM


def _tc_sum_body(x_ref, out_ref, acc_ref):
    i = pl.program_id(0)

    @pl.when(i == 0)
    def _():
        acc_ref[...] = jnp.zeros_like(acc_ref)

    acc_ref[...] += jnp.sum(x_ref[...].reshape(_BM // 8, 8, _N), axis=0)

    @pl.when(i == _GRID - 1)
    def _():
        out_ref[0] = jnp.sum(acc_ref[...])


_tc_sum = pl.pallas_call(
    _tc_sum_body,
    grid=(_GRID,),
    in_specs=[pl.BlockSpec((_BM, _N), lambda i: (i, 0))],
    out_specs=pl.BlockSpec(memory_space=pltpu.SMEM),
    out_shape=jax.ShapeDtypeStruct((1,), jnp.float32),
    scratch_shapes=[pltpu.VMEM((8, _N), jnp.float32)],
)


def _combine_body(tot_ref, parts_ref, out_ref):
    pos_sum = jnp.sum(parts_ref[0])
    mat_sum = tot_ref[0] + jnp.sum(parts_ref[1])
    out_ref[0] = (mat_sum / (_N * _N)) - (pos_sum / _N) + _MARGIN


_combine = pl.pallas_call(
    _combine_body,
    in_specs=[
        pl.BlockSpec(memory_space=pltpu.SMEM),
        pl.BlockSpec((2, _NW, _L), lambda: (0, 0, 0)),
    ],
    out_specs=pl.BlockSpec(memory_space=pltpu.SMEM),
    out_shape=jax.ShapeDtypeStruct((1,), jnp.float32),
)


def kernel(match_pair, similarity_matrix):
    mp = match_pair.astype(jnp.int32)
    tflat = (similarity_matrix
             .reshape(2048, 8, 128, 128)
             .transpose(0, 2, 1, 3)
             .reshape(_N * _N))
    partials = _sc_gather(mp[0], mp[1], tflat)
    total = _tc_sum(similarity_matrix)
    loss = _combine(total, partials)
    return loss[0]


# speedup vs baseline: 1.0247x; 1.0104x over previous
import functools

import jax
import jax.numpy as jnp
from jax import lax
from jax.experimental import pallas as pl
from jax.experimental.pallas import tpu as pltpu
from jax.experimental.pallas import tpu_sc as plsc

_MARGIN = 0.5
_N = 16384
_NC = 2
_NS = 16
_L = 16
_NW = _NC * _NS
_PER_W = _N // _NW
_CH = 128
_NCH = _PER_W // _CH

_ROWS_SC = 768
_SC_E = _ROWS_SC * _N
_SC_BASE = (_N - _ROWS_SC) * _N
_PT = _SC_E // _NW
_CE = 16384
_NMC = _PT // _CE
_ACCS = 8


def _sc_body(rows_hbm, cols_hbm, tflat_hbm, out_hbm,
             rows_v, cols_v, idx_v, vals_v, acc_v, buf0, buf1,
             gsem, sem0, sem1):
    wid = lax.axis_index("s") * _NC + lax.axis_index("c")
    base = wid * _PER_W
    pltpu.sync_copy(rows_hbm.at[pl.ds(base, _PER_W)], rows_v)
    pltpu.sync_copy(cols_hbm.at[pl.ds(base, _PER_W)], cols_v)
    for c in range(_NCH):
        for j in range(_CH // _L):
            o = c * _CH + j * _L
            r = rows_v[pl.ds(o, _L)]
            q = cols_v[pl.ds(o, _L)]
            f = (lax.shift_right_logical(r, 3) * 131072
                 + lax.shift_right_logical(q, 7) * 1024
                 + lax.bitwise_and(r, 7) * 128
                 + lax.bitwise_and(q, 127))
            idx_v[c, pl.ds(j * _L, _L)] = f
    copies = [
        pltpu.async_copy(tflat_hbm.at[idx_v.at[c]], vals_v.at[c], gsem)
        for c in range(_NCH)
    ]
    for cp in copies:
        cp.wait()
    acc = jnp.zeros((_L,), jnp.float32)
    for c in range(_NCH):
        for j in range(_CH // _L):
            acc = acc + vals_v[c, pl.ds(j * _L, _L)]
    acc_v[...] = acc
    pltpu.sync_copy(acc_v, out_hbm.at[0, wid])

    mbase = _SC_BASE + wid * _PT

    def _chunk_sum(buf, accs):
        def inner(j, accs):
            o = j * (_ACCS * _L)
            return tuple(
                accs[i] + buf[pl.ds(o + i * _L, _L)] for i in range(_ACCS)
            )
        return lax.fori_loop(0, _CE // (_ACCS * _L), inner, accs)

    def outer(kk, accs):
        pltpu.make_async_copy(tflat_hbm.at[pl.ds(0, _CE)], buf0, sem0).wait()

        @pl.when(kk * 2 + 1 < _NMC)
        def _():
            pltpu.async_copy(
                tflat_hbm.at[pl.ds(mbase + (kk * 2 + 1) * _CE, _CE)], buf1, sem1)

        accs = _chunk_sum(buf0, accs)
        pltpu.make_async_copy(tflat_hbm.at[pl.ds(0, _CE)], buf1, sem1).wait()

        @pl.when(kk < _NMC // 2 - 1)
        def _():
            pltpu.async_copy(
                tflat_hbm.at[pl.ds(mbase + (kk * 2 + 2) * _CE, _CE)], buf0, sem0)

        accs = _chunk_sum(buf1, accs)
        return accs

    pltpu.async_copy(tflat_hbm.at[pl.ds(mbase, _CE)], buf0, sem0)
    accs0 = tuple(jnp.zeros((_L,), jnp.float32) for _ in range(_ACCS))
    accs = lax.fori_loop(0, _NMC // 2, outer, accs0)
    macc = accs[0]
    for i in range(1, _ACCS):
        macc = macc + accs[i]
    acc_v[...] = macc
    pltpu.sync_copy(acc_v, out_hbm.at[1, wid])


_sc_gather = functools.partial(
    pl.kernel,
    out_type=jax.ShapeDtypeStruct((2, _NW, _L), jnp.float32),
    mesh=plsc.VectorSubcoreMesh(core_axis_name="c", subcore_axis_name="s"),
    scratch_types=[
        pltpu.VMEM((_PER_W,), jnp.int32),
        pltpu.VMEM((_PER_W,), jnp.int32),
        pltpu.VMEM((_NCH, _CH), jnp.int32),
        pltpu.VMEM((_NCH, _CH), jnp.float32),
        pltpu.VMEM((_L,), jnp.float32),
        pltpu.VMEM((_CE,), jnp.float32),
        pltpu.VMEM((_CE,), jnp.float32),
        pltpu.SemaphoreType.DMA,
        pltpu.SemaphoreType.DMA,
        pltpu.SemaphoreType.DMA,
    ],
)(_sc_body)


_BM = 256
_TC_ROWS = _N - _ROWS_SC
_GRID = _TC_ROWS // _BM


def _tc_sum_body(x_ref, out_ref, acc_ref):
    i = pl.program_id(0)

    @pl.when(i == 0)
    def _():
        acc_ref[...] = jnp.zeros_like(acc_ref)

    acc_ref[...] += jnp.sum(x_ref[...].reshape(_BM // 8, 8, _N), axis=0)

    @pl.when(i == _GRID - 1)
    def _():
        out_ref[0] = jnp.sum(acc_ref[...])


_tc_sum = pl.pallas_call(
    _tc_sum_body,
    grid=(_GRID,),
    in_specs=[pl.BlockSpec((_BM, _N), lambda i: (i, 0))],
    out_specs=pl.BlockSpec(memory_space=pltpu.SMEM),
    out_shape=jax.ShapeDtypeStruct((1,), jnp.float32),
    scratch_shapes=[pltpu.VMEM((8, _N), jnp.float32)],
)


def _combine_body(tot_ref, parts_ref, out_ref):
    pos_sum = jnp.sum(parts_ref[0])
    mat_sum = tot_ref[0] + jnp.sum(parts_ref[1])
    out_ref[0] = (mat_sum / (_N * _N)) - (pos_sum / _N) + _MARGIN


_combine = pl.pallas_call(
    _combine_body,
    in_specs=[
        pl.BlockSpec(memory_space=pltpu.SMEM),
        pl.BlockSpec((2, _NW, _L), lambda: (0, 0, 0)),
    ],
    out_specs=pl.BlockSpec(memory_space=pltpu.SMEM),
    out_shape=jax.ShapeDtypeStruct((1,), jnp.float32),
)


def kernel(match_pair, similarity_matrix):
    mp = match_pair.astype(jnp.int32)
    tflat = (similarity_matrix
             .reshape(2048, 8, 128, 128)
             .transpose(0, 2, 1, 3)
             .reshape(_N * _N))
    partials = _sc_gather(mp[0], mp[1], tflat)
    total = _tc_sum(similarity_matrix)
    loss = _combine(total, partials)
    return loss[0]
